# baseline (speedup 1.0000x reference)
import functools

import jax
import jax.numpy as jnp
from jax import lax
from jax.experimental import pallas as pl
from jax.experimental.pallas import tpu as pltpu
from jax.experimental.pallas import tpu_sc as plsc

V, D, B = 100000, 64, 4096
DA = 128
TM, TN = 512, 512
NI, NJ = B // TM, B // TN

_NC, _NS = 2, 16
NW = _NC * _NS
BPW = B // NW


@functools.cache
def _sc_gather_fn():
    mesh = plsc.VectorSubcoreMesh(core_axis_name="c", subcore_axis_name="s")

    @functools.partial(
        pl.kernel,
        mesh=mesh,
        out_type=[
            jax.ShapeDtypeStruct((B, D), jnp.float32),
            jax.ShapeDtypeStruct((B, D), jnp.float32),
            jax.ShapeDtypeStruct((B,), jnp.float32),
            jax.ShapeDtypeStruct((B,), jnp.float32),
        ],
        scratch_types=[
            pltpu.VMEM((BPW,), jnp.int32),
            pltpu.VMEM((BPW, D), jnp.float32),
            pltpu.VMEM((BPW, D), jnp.float32),
            pltpu.VMEM((BPW,), jnp.float32),
            pltpu.VMEM((BPW,), jnp.float32),
            pltpu.SemaphoreType.DMA,
        ],
        compiler_params=pltpu.CompilerParams(use_tc_tiling_on_sc=False),
    )
    def _sc_gather(idx_hbm, emb_hbm, embt_hbm, bias_hbm, biast_hbm,
                   w_hbm, wt_hbm, b_hbm, bt_hbm,
                   idx_v, w_v, wt_v, b_v, bt_v, sem):
        wid = lax.axis_index("s") * _NC + lax.axis_index("c")
        base = wid * BPW
        pltpu.sync_copy(idx_hbm.at[pl.ds(base, BPW)], idx_v)
        cps = [
            pltpu.async_copy(emb_hbm.at[idx_v], w_v, sem),
            pltpu.async_copy(embt_hbm.at[idx_v], wt_v, sem),
            pltpu.async_copy(bias_hbm.at[idx_v], b_v, sem),
            pltpu.async_copy(biast_hbm.at[idx_v], bt_v, sem),
        ]
        for cp in cps:
            cp.wait()
        pltpu.sync_copy(w_v, w_hbm.at[pl.ds(base, BPW)])
        pltpu.sync_copy(wt_v, wt_hbm.at[pl.ds(base, BPW)])
        pltpu.sync_copy(b_v, b_hbm.at[pl.ds(base, BPW)])
        pltpu.sync_copy(bt_v, bt_hbm.at[pl.ds(base, BPW)])

    return _sc_gather


def _loss_body(wa_ref, wb_ref, logx_ref, wts_ref, out_ref, acc_ref):
    i = pl.program_id(0)
    j = pl.program_id(1)

    @pl.when((i == 0) & (j == 0))
    def _init():
        acc_ref[0] = 0.0

    a = wa_ref[pl.ds(pl.multiple_of(i * TM, TM), TM), :]
    bt = wb_ref[pl.ds(pl.multiple_of(j * TN, TN), TN), :]
    m = lax.dot_general(a, bt, (((1,), (1,)), ((), ())),
                        preferred_element_type=jnp.float32)
    r = m - logx_ref[...]
    acc_ref[0] += jnp.sum(wts_ref[...] * r * r)

    @pl.when((i == NI - 1) & (j == NJ - 1))
    def _fin():
        out_ref[0, 0] = acc_ref[0] * (1.0 / (B * B))


def kernel(indices, logx, weights, emb, emb_tilde, bias, bias_tilde):
    idx = indices.astype(jnp.int32)
    w, wt, b, bt = _sc_gather_fn()(idx, emb, emb_tilde,
                                   bias.reshape(V), bias_tilde.reshape(V))
    ones = jnp.ones((B, 1), jnp.float32)
    zpad = jnp.zeros((B, DA - D - 2), jnp.float32)
    wa = jnp.concatenate([w, b[:, None], ones, zpad], axis=1)
    wb = jnp.concatenate([wt, ones, bt[:, None], zpad], axis=1)

    out = pl.pallas_call(
        _loss_body,
        grid=(NI, NJ),
        in_specs=[
            pl.BlockSpec((B, DA), lambda i, j: (0, 0)),
            pl.BlockSpec((B, DA), lambda i, j: (0, 0)),
            pl.BlockSpec((TM, TN), lambda i, j: (i, j)),
            pl.BlockSpec((TM, TN), lambda i, j: (i, j)),
        ],
        out_specs=pl.BlockSpec((1, 1), lambda i, j: (0, 0),
                               memory_space=pltpu.SMEM),
        out_shape=jax.ShapeDtypeStruct((1, 1), jnp.float32),
        scratch_shapes=[pltpu.SMEM((1,), jnp.float32)],
        compiler_params=pltpu.CompilerParams(
            dimension_semantics=("arbitrary", "arbitrary")),
    )(wa, wb, logx, weights)
    return out[0, 0]


# speedup vs baseline: 1.1554x; 1.1554x over previous
import functools

import jax
import jax.numpy as jnp
from jax import lax
from jax.experimental import pallas as pl
from jax.experimental.pallas import tpu as pltpu
from jax.experimental.pallas import tpu_sc as plsc

V, D, B = 100000, 64, 4096
DA = 128
TM, TN = 1024, 1024
NI, NJ = B // TM, B // TN

_NC, _NS = 2, 16
NW = _NC * _NS
BPW = B // NW


@functools.cache
def _sc_gather_fn():
    mesh = plsc.VectorSubcoreMesh(core_axis_name="c", subcore_axis_name="s")

    @functools.partial(
        pl.kernel,
        mesh=mesh,
        out_type=[
            jax.ShapeDtypeStruct((B, D), jnp.float32),
            jax.ShapeDtypeStruct((B, D), jnp.float32),
            jax.ShapeDtypeStruct((B,), jnp.float32),
            jax.ShapeDtypeStruct((B,), jnp.float32),
        ],
        scratch_types=[
            pltpu.VMEM((BPW,), jnp.int32),
            pltpu.VMEM((BPW, D), jnp.float32),
            pltpu.VMEM((BPW, D), jnp.float32),
            pltpu.VMEM((BPW,), jnp.float32),
            pltpu.VMEM((BPW,), jnp.float32),
            pltpu.SemaphoreType.DMA,
        ],
        compiler_params=pltpu.CompilerParams(use_tc_tiling_on_sc=False),
    )
    def _sc_gather(idx_hbm, emb_hbm, embt_hbm, bias_hbm, biast_hbm,
                   w_hbm, wt_hbm, b_hbm, bt_hbm,
                   idx_v, w_v, wt_v, b_v, bt_v, sem):
        wid = lax.axis_index("s") * _NC + lax.axis_index("c")
        base = wid * BPW
        pltpu.sync_copy(idx_hbm.at[pl.ds(base, BPW)], idx_v)
        cps = [
            pltpu.async_copy(emb_hbm.at[idx_v], w_v, sem),
            pltpu.async_copy(embt_hbm.at[idx_v], wt_v, sem),
            pltpu.async_copy(bias_hbm.at[idx_v], b_v, sem),
            pltpu.async_copy(biast_hbm.at[idx_v], bt_v, sem),
        ]
        for cp in cps:
            cp.wait()
        pltpu.sync_copy(w_v, w_hbm.at[pl.ds(base, BPW)])
        pltpu.sync_copy(wt_v, wt_hbm.at[pl.ds(base, BPW)])
        pltpu.sync_copy(b_v, b_hbm.at[pl.ds(base, BPW)])
        pltpu.sync_copy(bt_v, bt_hbm.at[pl.ds(base, BPW)])

    return _sc_gather


def _loss_body(wa_ref, wb_ref, logx_ref, wts_ref, out_ref, acc_ref):
    i = pl.program_id(0)
    j = pl.program_id(1)

    @pl.when((i == 0) & (j == 0))
    def _init():
        acc_ref[0] = 0.0

    a = wa_ref[pl.ds(pl.multiple_of(i * TM, TM), TM), :]
    bt = wb_ref[pl.ds(pl.multiple_of(j * TN, TN), TN), :]
    m = lax.dot_general(a, bt, (((1,), (1,)), ((), ())),
                        preferred_element_type=jnp.float32)
    r = m - logx_ref[...]
    acc_ref[0] += jnp.sum(wts_ref[...] * r * r)

    @pl.when((i == NI - 1) & (j == NJ - 1))
    def _fin():
        out_ref[0, 0] = acc_ref[0] * (1.0 / (B * B))


def kernel(indices, logx, weights, emb, emb_tilde, bias, bias_tilde):
    idx = indices.astype(jnp.int32)
    w, wt, b, bt = _sc_gather_fn()(idx, emb, emb_tilde,
                                   bias.reshape(V), bias_tilde.reshape(V))
    ones = jnp.ones((B, 1), jnp.float32)
    zpad = jnp.zeros((B, DA - D - 2), jnp.float32)
    wa = jnp.concatenate([w, b[:, None], ones, zpad], axis=1)
    wb = jnp.concatenate([wt, ones, bt[:, None], zpad], axis=1)

    out = pl.pallas_call(
        _loss_body,
        grid=(NI, NJ),
        in_specs=[
            pl.BlockSpec((B, DA), lambda i, j: (0, 0)),
            pl.BlockSpec((B, DA), lambda i, j: (0, 0)),
            pl.BlockSpec((TM, TN), lambda i, j: (i, j)),
            pl.BlockSpec((TM, TN), lambda i, j: (i, j)),
        ],
        out_specs=pl.BlockSpec((1, 1), lambda i, j: (0, 0),
                               memory_space=pltpu.SMEM),
        out_shape=jax.ShapeDtypeStruct((1, 1), jnp.float32),
        scratch_shapes=[pltpu.SMEM((1,), jnp.float32)],
        compiler_params=pltpu.CompilerParams(
            dimension_semantics=("arbitrary", "arbitrary")),
    )(wa, wb, logx, weights)
    return out[0, 0]


# speedup vs baseline: 1.3432x; 1.1626x over previous
import functools

import jax
import jax.numpy as jnp
from jax import lax
from jax.experimental import pallas as pl
from jax.experimental.pallas import tpu as pltpu
from jax.experimental.pallas import tpu_sc as plsc

V, D, B = 100000, 64, 4096
DA = 72
TM, TN = 1024, 1024
NI, NJ = B // TM, B // TN

_NC, _NS = 2, 16
NW = _NC * _NS
BPW = B // NW
_L = 16


@functools.cache
def _sc_gather_fn():
    mesh = plsc.VectorSubcoreMesh(core_axis_name="c", subcore_axis_name="s")

    @functools.partial(
        pl.kernel,
        mesh=mesh,
        out_type=[
            jax.ShapeDtypeStruct((D, B), jnp.float32),
            jax.ShapeDtypeStruct((D, B), jnp.float32),
            jax.ShapeDtypeStruct((B,), jnp.float32),
            jax.ShapeDtypeStruct((B,), jnp.float32),
        ],
        scratch_types=[
            pltpu.VMEM((BPW,), jnp.int32),
            pltpu.VMEM((D, BPW), jnp.int32),
            pltpu.VMEM((D, BPW), jnp.float32),
            pltpu.VMEM((D, BPW), jnp.float32),
            pltpu.VMEM((BPW,), jnp.float32),
            pltpu.VMEM((BPW,), jnp.float32),
            pltpu.SemaphoreType.DMA,
            pltpu.SemaphoreType.DMA,
        ],
        compiler_params=pltpu.CompilerParams(use_tc_tiling_on_sc=False),
    )
    def _sc_gather(idx_hbm, emb1_hbm, emb2_hbm, bias_hbm, biast_hbm,
                   waT_hbm, wbT_hbm, b_hbm, bt_hbm,
                   idx_v, offs_v, wa_v, wb_v, b_v, bt_v, sem, semb):
        wid = lax.axis_index("s") * _NC + lax.axis_index("c")
        base = wid * BPW
        pltpu.sync_copy(idx_hbm.at[pl.ds(base, BPW)], idx_v)
        cpb1 = pltpu.async_copy(bias_hbm.at[idx_v], b_v, semb)
        cpb2 = pltpu.async_copy(biast_hbm.at[idx_v], bt_v, semb)
        for g in range(BPW // _L):
            v16 = idx_v[pl.ds(g * _L, _L)]
            for j in range(D):
                offs_v[j, pl.ds(g * _L, _L)] = v16 + (j * V)
        rounds = []
        RJ = 8
        for r in range(D // RJ):
            cps = []
            for j in range(r * RJ, (r + 1) * RJ):
                cps.append(pltpu.async_copy(
                    emb1_hbm.at[offs_v.at[j]], wa_v.at[j], sem))
                cps.append(pltpu.async_copy(
                    emb2_hbm.at[offs_v.at[j]], wb_v.at[j], sem))
            rounds.append(cps)
            if r > 0:
                for cp in rounds[r - 1]:
                    cp.wait()
        for cp in rounds[-1]:
            cp.wait()
        pltpu.sync_copy(wa_v, waT_hbm.at[:, pl.ds(base, BPW)])
        pltpu.sync_copy(wb_v, wbT_hbm.at[:, pl.ds(base, BPW)])
        cpb1.wait()
        cpb2.wait()
        pltpu.sync_copy(b_v, b_hbm.at[pl.ds(base, BPW)])
        pltpu.sync_copy(bt_v, bt_hbm.at[pl.ds(base, BPW)])

    return _sc_gather


def _loss_body(wa_ref, wb_ref, logx_ref, wts_ref, out_ref, acc_ref):
    i = pl.program_id(0)
    j = pl.program_id(1)

    @pl.when((i == 0) & (j == 0))
    def _init():
        acc_ref[0] = 0.0

    m = lax.dot_general(wa_ref[...], wb_ref[...], (((0,), (0,)), ((), ())),
                        preferred_element_type=jnp.float32)
    r = m - logx_ref[...]
    acc_ref[0] += jnp.sum(wts_ref[...] * r * r)

    @pl.when((i == NI - 1) & (j == NJ - 1))
    def _fin():
        out_ref[0, 0] = acc_ref[0] * (1.0 / (B * B))


def kernel(indices, logx, weights, emb, emb_tilde, bias, bias_tilde):
    idx = indices.astype(jnp.int32)
    waT, wbT, b, bt = _sc_gather_fn()(idx, emb.T.reshape(V * D),
                                      emb_tilde.T.reshape(V * D),
                                      bias.reshape(V), bias_tilde.reshape(V))
    ones = jnp.ones((1, B), jnp.float32)
    zpad = jnp.zeros((DA - D - 2, B), jnp.float32)
    wa = jnp.concatenate([waT, b[None, :], ones, zpad], axis=0)
    wb = jnp.concatenate([wbT, ones, bt[None, :], zpad], axis=0)

    out = pl.pallas_call(
        _loss_body,
        grid=(NI, NJ),
        in_specs=[
            pl.BlockSpec((DA, TM), lambda i, j: (0, i)),
            pl.BlockSpec((DA, TN), lambda i, j: (0, j)),
            pl.BlockSpec((TM, TN), lambda i, j: (i, j)),
            pl.BlockSpec((TM, TN), lambda i, j: (i, j)),
        ],
        out_specs=pl.BlockSpec((1, 1), lambda i, j: (0, 0),
                               memory_space=pltpu.SMEM),
        out_shape=jax.ShapeDtypeStruct((1, 1), jnp.float32),
        scratch_shapes=[pltpu.SMEM((1,), jnp.float32)],
        compiler_params=pltpu.CompilerParams(
            dimension_semantics=("arbitrary", "arbitrary")),
    )(wa, wb, logx, weights)
    return out[0, 0]


# speedup vs baseline: 1.4280x; 1.0631x over previous
import functools

import jax
import jax.numpy as jnp
from jax import lax
from jax.experimental import pallas as pl
from jax.experimental.pallas import tpu as pltpu
from jax.experimental.pallas import tpu_sc as plsc

V, D, B = 100000, 64, 4096
DA = 72
TM, TN = 1024, 2048
NI, NJ = B // TM, B // TN

_NC, _NS = 2, 16
NW = _NC * _NS
BPW = B // NW
_L = 16


@functools.cache
def _sc_gather_fn():
    mesh = plsc.VectorSubcoreMesh(core_axis_name="c", subcore_axis_name="s")

    @functools.partial(
        pl.kernel,
        mesh=mesh,
        out_type=[
            jax.ShapeDtypeStruct((D, B), jnp.float32),
            jax.ShapeDtypeStruct((B,), jnp.float32),
        ],
        scratch_types=[
            pltpu.VMEM((BPW,), jnp.int32),
            pltpu.VMEM((D, BPW), jnp.int32),
            pltpu.VMEM((D, BPW), jnp.float32),
            pltpu.VMEM((BPW,), jnp.float32),
            pltpu.SemaphoreType.DMA,
            pltpu.SemaphoreType.DMA,
        ],
        compiler_params=pltpu.CompilerParams(use_tc_tiling_on_sc=False),
    )
    def _sc_gather(idx_hbm, emb1_hbm, bias_hbm,
                   waT_hbm, b_hbm,
                   idx_v, offs_v, wa_v, b_v, sem, semb):
        wid = lax.axis_index("s") * _NC + lax.axis_index("c")
        base = wid * BPW
        pltpu.sync_copy(idx_hbm.at[pl.ds(base, BPW)], idx_v)
        cpb1 = pltpu.async_copy(bias_hbm.at[idx_v], b_v, semb)
        for g in range(BPW // _L):
            v16 = idx_v[pl.ds(g * _L, _L)]
            for j in range(D):
                offs_v[j, pl.ds(g * _L, _L)] = v16 + (j * V)
        rounds = []
        RJ = 8
        for r in range(D // RJ):
            cps = []
            for j in range(r * RJ, (r + 1) * RJ):
                cps.append(pltpu.async_copy(
                    emb1_hbm.at[offs_v.at[j]], wa_v.at[j], sem))
            rounds.append(cps)
            if r > 0:
                for cp in rounds[r - 1]:
                    cp.wait()
        for cp in rounds[-1]:
            cp.wait()
        pltpu.sync_copy(wa_v, waT_hbm.at[:, pl.ds(base, BPW)])
        cpb1.wait()
        pltpu.sync_copy(b_v, b_hbm.at[pl.ds(base, BPW)])

    return _sc_gather


def _loss_body(wa_ref, wb_ref, logx_ref, wts_ref, out_ref, acc_ref):
    i = pl.program_id(0)
    j = pl.program_id(1)

    @pl.when((i == 0) & (j == 0))
    def _init():
        acc_ref[0] = 0.0

    m = lax.dot_general(wa_ref[...], wb_ref[...], (((0,), (0,)), ((), ())),
                        preferred_element_type=jnp.float32)
    r = m - logx_ref[...]
    acc_ref[0] += jnp.sum(wts_ref[...] * r * r)

    @pl.when((i == NI - 1) & (j == NJ - 1))
    def _fin():
        out_ref[0, 0] = acc_ref[0] * (1.0 / (B * B))


def kernel(indices, logx, weights, emb, emb_tilde, bias, bias_tilde):
    idx = indices.astype(jnp.int32)
    gather = _sc_gather_fn()
    waT, b = gather(idx, emb.T.reshape(V * D), bias.reshape(V))
    wbT, bt = gather(idx, emb_tilde.T.reshape(V * D), bias_tilde.reshape(V))
    ones = jnp.ones((1, B), jnp.float32)
    zpad = jnp.zeros((DA - D - 2, B), jnp.float32)
    wa = jnp.concatenate([waT, b[None, :], ones, zpad], axis=0)
    wb = jnp.concatenate([wbT, ones, bt[None, :], zpad], axis=0)

    out = pl.pallas_call(
        _loss_body,
        grid=(NI, NJ),
        in_specs=[
            pl.BlockSpec((DA, TM), lambda i, j: (0, i)),
            pl.BlockSpec((DA, TN), lambda i, j: (0, j)),
            pl.BlockSpec((TM, TN), lambda i, j: (i, j)),
            pl.BlockSpec((TM, TN), lambda i, j: (i, j)),
        ],
        out_specs=pl.BlockSpec((1, 1), lambda i, j: (0, 0),
                               memory_space=pltpu.SMEM),
        out_shape=jax.ShapeDtypeStruct((1, 1), jnp.float32),
        scratch_shapes=[pltpu.SMEM((1,), jnp.float32)],
        compiler_params=pltpu.CompilerParams(
            dimension_semantics=("arbitrary", "arbitrary")),
    )(wa, wb, logx, weights)
    return out[0, 0]
